# baseline (speedup 1.0000x reference)
import functools

import jax
import jax.numpy as jnp
from jax import lax
from jax.experimental import pallas as pl
from jax.experimental.pallas import tpu as pltpu
from jax.experimental.pallas import tpu_sc as plsc

_GRP = 16
_NBUF = 2


@functools.lru_cache(maxsize=None)
def _make_gather(batch, hist, d):
    info = plsc.get_sparse_core_info()
    num_cores, num_subcores = info.num_cores, info.num_subcores
    num_workers = num_cores * num_subcores
    ent_w = batch // num_workers
    n_grp = ent_w // _GRP
    assert ent_w * num_workers == batch and n_grp * _GRP == ent_w
    assert n_grp % _NBUF == 0

    mesh = plsc.VectorSubcoreMesh(core_axis_name="c", subcore_axis_name="s")

    @functools.partial(
        pl.kernel,
        out_type=jax.ShapeDtypeStruct((batch, hist, d), jnp.float32),
        mesh=mesh,
        scratch_types=[
            pltpu.VMEM((ent_w, hist), jnp.int32),
            pltpu.VMEM((_NBUF, _GRP, hist, d), jnp.float32),
        ]
        + [pltpu.SemaphoreType.DMA] * (2 * _NBUF),
        compiler_params=pltpu.CompilerParams(use_tc_tiling_on_sc=False),
    )
    def gather_kernel(table_hbm, idx_hbm, out_hbm, idx_v, rows_v, *sems):
        gsem = sems[:_NBUF]
        ssem = sems[_NBUF:]
        wid = lax.axis_index("s") * num_cores + lax.axis_index("c")
        e0 = wid * ent_w
        pltpu.sync_copy(idx_hbm.at[pl.ds(e0, ent_w)], idx_v)

        def grp_gathers(j, b):
            return [
                pltpu.make_async_copy(
                    table_hbm.at[idx_v.at[j * _GRP + t]],
                    rows_v.at[b, t],
                    gsem[b],
                )
                for t in range(_GRP)
            ]

        def grp_store(j, b):
            return pltpu.make_async_copy(
                rows_v.at[b],
                out_hbm.at[pl.ds(e0 + j * _GRP, _GRP)],
                ssem[b],
            )

        for b in range(_NBUF):
            for c in grp_gathers(b, b):
                c.start()

        def outer(g, carry):
            for b in range(_NBUF):
                j = g * _NBUF + b
                for c in grp_gathers(j, b):
                    c.wait()
                grp_store(j, b).start()
                bp = (b - 1) % _NBUF
                jp = j - 1 + _NBUF

                @pl.when((j >= 1) & (jp < n_grp))
                def _():
                    grp_store(jp - _NBUF, bp).wait()
                    for c in grp_gathers(jp, bp):
                        c.start()

            return carry

        lax.fori_loop(0, n_grp // _NBUF, outer, 0)
        for b in range(_NBUF):
            grp_store(n_grp - _NBUF + b, b).wait()

    return gather_kernel


def kernel(matrix, node_seq_id, G=0):
    batch, hist = node_seq_id.shape
    d = matrix.shape[1]
    return _make_gather(batch, hist, d)(matrix, node_seq_id)
